# baseline (speedup 1.0000x reference)
import jax
import jax.numpy as jnp
from jax.experimental import pallas as pl
from jax.experimental.pallas import tpu as pltpu

_TB = 32
_NC = 4


def _vq_onehot_body(x_ref, cb_ref, out_ref, c2_ref, cbm2_ref):
    K = cb_ref.shape[0]
    KC = K // _NC
    i = pl.program_id(0)

    @pl.when(i == 0)
    def _():
        cb = cb_ref[...]
        c2_ref[...] = jnp.sum(cb * cb, axis=-1)[None, :]
        cbm2_ref[...] = cb * (-2.0)

    x = x_ref[...]
    z2 = jnp.sum(x * x, axis=-1, keepdims=True)
    crosses = []
    minval = None
    for c in range(_NC):
        cross_c = jax.lax.dot_general(
            x, cbm2_ref[c * KC:(c + 1) * KC, :], (((1,), (1,)), ((), ())),
            preferred_element_type=jnp.float32)
        crosses.append(cross_c)
        m_c = jnp.min(z2 + cross_c + c2_ref[:, c * KC:(c + 1) * KC],
                      axis=-1, keepdims=True)
        minval = m_c if minval is None else jnp.minimum(minval, m_c)
    codes = None
    for c in range(_NC):
        lane = jax.lax.broadcasted_iota(jnp.int32, crosses[c].shape, 1)
        dist_c = (crosses[c] + z2) + c2_ref[:, c * KC:(c + 1) * KC]
        idx_c = jnp.min(jnp.where(dist_c == minval, lane + (c * KC), K),
                        axis=-1, keepdims=True)
        codes = idx_c if codes is None else jnp.minimum(codes, idx_c)
    B = out_ref.shape[0]
    kiota = jax.lax.broadcasted_iota(jnp.int32, (B, K), 1)
    for t in range(_TB):
        target = codes[t * B:(t + 1) * B]
        out_ref[:, t * K:(t + 1) * K] = (target == kiota).astype(out_ref.dtype)


def kernel(input, codebook):
    B, T, D = input.shape
    K = codebook.shape[0]
    x = input.transpose(1, 0, 2).reshape(T * B, D)
    onehot = pl.pallas_call(
        _vq_onehot_body,
        grid=(T // _TB,),
        in_specs=[
            pl.BlockSpec((_TB * B, D), lambda i: (i, 0)),
            pl.BlockSpec((K, D), lambda i: (0, 0)),
        ],
        out_specs=pl.BlockSpec((B, _TB * K), lambda i: (0, i)),
        out_shape=jax.ShapeDtypeStruct((B, T * K), jnp.int32),
        scratch_shapes=[pltpu.VMEM((1, K), jnp.float32),
                        pltpu.VMEM((K, D), jnp.float32)],
    )(x, codebook)
    return onehot.astype(jnp.int64)


# speedup vs baseline: 1.1085x; 1.1085x over previous
import jax
import jax.numpy as jnp
from jax.experimental import pallas as pl
from jax.experimental.pallas import tpu as pltpu

_TB = 32


def _vq_onehot_body(x_ref, cb_ref, out_ref, c2_ref):
    K = cb_ref.shape[0]
    i = pl.program_id(0)

    @pl.when(i == 0)
    def _():
        cb = cb_ref[...]
        c2_ref[...] = jnp.sum(cb * cb, axis=-1)[None, :]

    x = x_ref[...]
    z2 = jnp.sum(x * x, axis=-1, keepdims=True)
    cross = jax.lax.dot_general(
        x, cb_ref[...], (((1,), (1,)), ((), ())),
        preferred_element_type=jnp.float32)
    dist = z2 - 2.0 * cross + c2_ref[...]
    minval = jnp.min(dist, axis=-1, keepdims=True)
    lane = jax.lax.broadcasted_iota(jnp.int32, dist.shape, 1)
    codes = jnp.min(jnp.where(dist == minval, lane, K),
                    axis=-1, keepdims=True)
    B = out_ref.shape[0]
    kiota = jax.lax.broadcasted_iota(jnp.int32, (B, K), 1)
    for t in range(_TB):
        target = codes[t * B:(t + 1) * B]
        out_ref[:, t * K:(t + 1) * K] = (target == kiota).astype(out_ref.dtype)


def kernel(input, codebook):
    B, T, D = input.shape
    K = codebook.shape[0]
    x = input.transpose(1, 0, 2).reshape(T * B, D)
    onehot = pl.pallas_call(
        _vq_onehot_body,
        grid=(T // _TB,),
        in_specs=[
            pl.BlockSpec((_TB * B, D), lambda i: (i, 0)),
            pl.BlockSpec((K, D), lambda i: (0, 0)),
        ],
        out_specs=pl.BlockSpec((B, _TB * K), lambda i: (0, i)),
        out_shape=jax.ShapeDtypeStruct((B, T * K), jnp.int32),
        scratch_shapes=[pltpu.VMEM((1, K), jnp.float32)],
    )(x, codebook)
    return onehot.astype(jnp.int64)
